# baseline (speedup 1.0000x reference)
import functools

import jax
import jax.numpy as jnp
from jax import lax
from jax.experimental import pallas as pl
from jax.experimental.pallas import tpu as pltpu
from jax.experimental.pallas import tpu_sc as plsc

B = 16384
D = 64
NROW = 1000000
NC = 2
NS = 16
NW = NC * NS
BPW = B // NW
CHUNK = 16
NCHUNK = BPW // CHUNK
CSLOT = CHUNK // 8
NPAIR = NCHUNK // 2


def _issue_chunk(tab, ids_v, buf, k0, sem):
    r16v = (ids_v[pl.ds(k0, 16)] >> 4) << 4
    for jl in range(CHUNK):
        r16 = pl.multiple_of(r16v[jl], 16)
        pltpu.async_copy(
            tab.at[:, :, pl.ds(r16, 16)],
            buf.at[jl >> 3, :, :, pl.ds((jl & 7) * 16, 16)],
            sem)


def _drain_chunk(tab, buf, sem):
    for sl in range(CSLOT):
        pltpu.make_async_copy(tab.at[:, :, pl.ds(0, 128)], buf.at[sl],
                              sem).wait()


def _compute_chunk(uids_v, iids_v, ubuf, ibuf, scores, k0, lane):
    slot_vec = lane >> 3
    rho_u = uids_v[pl.ds(k0, 16)] & 15
    rho_i = iids_v[pl.ds(k0, 16)] & 15
    colu = (lane & 7) * 16 + rho_u
    coli = (lane & 7) * 16 + rho_i
    acc = jnp.zeros((16,), jnp.float32)
    for d in range(D):
        bvec = jnp.full((16,), d >> 3, jnp.int32)
        svec = jnp.full((16,), d & 7, jnp.int32)
        uv = plsc.load_gather(ubuf, [slot_vec, bvec, svec, colu])
        iv = plsc.load_gather(ibuf, [slot_vec, bvec, svec, coli])
        acc += uv * iv
    scores[pl.ds(k0, 16)] = acc


def _body(uids_hbm, iids_hbm, utab_hbm, itab_hbm, out_hbm,
          uids_v, iids_v, ubufA, ibufA, ubufB, ibufB, scores,
          usemA, isemA, usemB, isemB):
    wid = lax.axis_index("s") * NC + lax.axis_index("c")
    base = wid * BPW

    pltpu.sync_copy(uids_hbm.at[pl.ds(base, BPW)], uids_v.at[pl.ds(0, BPW)])
    pltpu.sync_copy(iids_hbm.at[pl.ds(base, BPW)], iids_v.at[pl.ds(0, BPW)])

    lane = lax.iota(jnp.int32, 16)

    _issue_chunk(utab_hbm, uids_v, ubufA, 0, usemA)
    _issue_chunk(itab_hbm, iids_v, ibufA, 0, isemA)

    def pair_step(m, carry):
        k0a = (2 * m) * CHUNK
        k0b = (2 * m + 1) * CHUNK
        _issue_chunk(utab_hbm, uids_v, ubufB, k0b, usemB)
        _issue_chunk(itab_hbm, iids_v, ibufB, k0b, isemB)
        _drain_chunk(utab_hbm, ubufA, usemA)
        _drain_chunk(itab_hbm, ibufA, isemA)
        _compute_chunk(uids_v, iids_v, ubufA, ibufA, scores, k0a, lane)

        @pl.when(m < NPAIR - 1)
        def _():
            k0n = (2 * m + 2) * CHUNK
            _issue_chunk(utab_hbm, uids_v, ubufA, k0n, usemA)
            _issue_chunk(itab_hbm, iids_v, ibufA, k0n, isemA)

        _drain_chunk(utab_hbm, ubufB, usemB)
        _drain_chunk(itab_hbm, ibufB, isemB)
        _compute_chunk(uids_v, iids_v, ubufB, ibufB, scores, k0b, lane)
        return carry

    lax.fori_loop(0, NPAIR, pair_step, 0)

    pltpu.sync_copy(scores, out_hbm.at[pl.ds(base, BPW)])


@functools.partial(
    pl.kernel,
    out_type=jax.ShapeDtypeStruct((B,), jnp.float32),
    mesh=plsc.VectorSubcoreMesh(core_axis_name="c", subcore_axis_name="s"),
    compiler_params=pltpu.CompilerParams(
        needs_layout_passes=False, use_tc_tiling_on_sc=True),
    scratch_types=[
        pltpu.VMEM((BPW + 16,), jnp.int32),
        pltpu.VMEM((BPW + 16,), jnp.int32),
        pltpu.VMEM((CSLOT, 8, 8, 128), jnp.float32),
        pltpu.VMEM((CSLOT, 8, 8, 128), jnp.float32),
        pltpu.VMEM((CSLOT, 8, 8, 128), jnp.float32),
        pltpu.VMEM((CSLOT, 8, 8, 128), jnp.float32),
        pltpu.VMEM((BPW,), jnp.float32),
        pltpu.SemaphoreType.DMA,
        pltpu.SemaphoreType.DMA,
        pltpu.SemaphoreType.DMA,
        pltpu.SemaphoreType.DMA,
    ],
)
def _two_tower_sc(uids_hbm, iids_hbm, utab_hbm, itab_hbm, out_hbm,
                  uids_v, iids_v, ubufA, ibufA, ubufB, ibufB, scores,
                  usemA, isemA, usemB, isemB):
    _body(uids_hbm, iids_hbm, utab_hbm, itab_hbm, out_hbm,
          uids_v, iids_v, ubufA, ibufA, ubufB, ibufB, scores,
          usemA, isemA, usemB, isemB)


@jax.jit
def kernel(user_ids, item_ids, user_table, item_table):
    utabt = user_table.T.reshape(8, 8, NROW)
    itabt = item_table.T.reshape(8, 8, NROW)
    return _two_tower_sc(user_ids.astype(jnp.int32),
                         item_ids.astype(jnp.int32), utabt, itabt)


# speedup vs baseline: 1.1276x; 1.1276x over previous
import functools

import jax
import jax.numpy as jnp
from jax import lax
from jax.experimental import pallas as pl
from jax.experimental.pallas import tpu as pltpu
from jax.experimental.pallas import tpu_sc as plsc

B = 16384
D = 64
NROW = 1000000
NC = 2
NS = 16
NW = NC * NS
BPW = B // NW
CHUNK = 32
NCHUNK = BPW // CHUNK
CSLOT = CHUNK // 16
NPAIR = NCHUNK // 2


def _issue_chunk(tab, ids_v, buf, k0, sem):
    def issue_group(g, carry):
        r8v = (ids_v[pl.ds(k0 + g * 16, 16)] >> 3) << 3
        for jl in range(16):
            r8 = pl.multiple_of(r8v[jl], 8)
            pltpu.async_copy(
                tab.at[:, :, pl.ds(r8, 8)],
                buf.at[g, :, :, pl.ds(jl * 8, 8)],
                sem)
        return carry

    lax.fori_loop(0, CHUNK // 16, issue_group, 0)


def _drain_chunk(tab, buf, sem):
    for sl in range(CSLOT):
        pltpu.make_async_copy(tab.at[:, :, pl.ds(0, 128)], buf.at[sl],
                              sem).wait()


def _compute_chunk(uids_v, iids_v, ubuf, ibuf, scores, k0, lane):
    for g in range(CHUNK // 16):
        slot_vec = jnp.full((16,), g, jnp.int32)
        rho_u = uids_v[pl.ds(k0 + g * 16, 16)] & 7
        rho_i = iids_v[pl.ds(k0 + g * 16, 16)] & 7
        colu = lane * 8 + rho_u
        coli = lane * 8 + rho_i
        acc = jnp.zeros((16,), jnp.float32)
        for d in range(D):
            bvec = jnp.full((16,), d >> 3, jnp.int32)
            svec = jnp.full((16,), d & 7, jnp.int32)
            uv = plsc.load_gather(ubuf, [slot_vec, bvec, svec, colu])
            iv = plsc.load_gather(ibuf, [slot_vec, bvec, svec, coli])
            acc += uv * iv
        scores[pl.ds(k0 + g * 16, 16)] = acc


def _body(uids_hbm, iids_hbm, utab_hbm, itab_hbm, out_hbm,
          uids_v, iids_v, ubufA, ibufA, ubufB, ibufB, scores,
          usemA, isemA, usemB, isemB):
    wid = lax.axis_index("s") * NC + lax.axis_index("c")
    base = wid * BPW

    pltpu.sync_copy(uids_hbm.at[pl.ds(base, BPW)], uids_v.at[pl.ds(0, BPW)])
    pltpu.sync_copy(iids_hbm.at[pl.ds(base, BPW)], iids_v.at[pl.ds(0, BPW)])

    lane = lax.iota(jnp.int32, 16)

    _issue_chunk(utab_hbm, uids_v, ubufA, 0, usemA)
    _issue_chunk(itab_hbm, iids_v, ibufA, 0, isemA)

    def pair_step(m, carry):
        k0a = (2 * m) * CHUNK
        k0b = (2 * m + 1) * CHUNK
        _issue_chunk(utab_hbm, uids_v, ubufB, k0b, usemB)
        _issue_chunk(itab_hbm, iids_v, ibufB, k0b, isemB)
        _drain_chunk(utab_hbm, ubufA, usemA)
        _drain_chunk(itab_hbm, ibufA, isemA)
        _compute_chunk(uids_v, iids_v, ubufA, ibufA, scores, k0a, lane)

        @pl.when(m < NPAIR - 1)
        def _():
            k0n = (2 * m + 2) * CHUNK
            _issue_chunk(utab_hbm, uids_v, ubufA, k0n, usemA)
            _issue_chunk(itab_hbm, iids_v, ibufA, k0n, isemA)

        _drain_chunk(utab_hbm, ubufB, usemB)
        _drain_chunk(itab_hbm, ibufB, isemB)
        _compute_chunk(uids_v, iids_v, ubufB, ibufB, scores, k0b, lane)
        return carry

    lax.fori_loop(0, NPAIR, pair_step, 0)

    pltpu.sync_copy(scores, out_hbm.at[pl.ds(base, BPW)])


@functools.partial(
    pl.kernel,
    out_type=jax.ShapeDtypeStruct((B,), jnp.float32),
    mesh=plsc.VectorSubcoreMesh(core_axis_name="c", subcore_axis_name="s"),
    compiler_params=pltpu.CompilerParams(
        needs_layout_passes=False, use_tc_tiling_on_sc=True),
    scratch_types=[
        pltpu.VMEM((BPW + 16,), jnp.int32),
        pltpu.VMEM((BPW + 16,), jnp.int32),
        pltpu.VMEM((CSLOT, 8, 8, 128), jnp.float32),
        pltpu.VMEM((CSLOT, 8, 8, 128), jnp.float32),
        pltpu.VMEM((CSLOT, 8, 8, 128), jnp.float32),
        pltpu.VMEM((CSLOT, 8, 8, 128), jnp.float32),
        pltpu.VMEM((BPW,), jnp.float32),
        pltpu.SemaphoreType.DMA,
        pltpu.SemaphoreType.DMA,
        pltpu.SemaphoreType.DMA,
        pltpu.SemaphoreType.DMA,
    ],
)
def _two_tower_sc(uids_hbm, iids_hbm, utab_hbm, itab_hbm, out_hbm,
                  uids_v, iids_v, ubufA, ibufA, ubufB, ibufB, scores,
                  usemA, isemA, usemB, isemB):
    _body(uids_hbm, iids_hbm, utab_hbm, itab_hbm, out_hbm,
          uids_v, iids_v, ubufA, ibufA, ubufB, ibufB, scores,
          usemA, isemA, usemB, isemB)


@jax.jit
def kernel(user_ids, item_ids, user_table, item_table):
    utabt = user_table.T.reshape(8, 8, NROW)
    itabt = item_table.T.reshape(8, 8, NROW)
    return _two_tower_sc(user_ids.astype(jnp.int32),
                         item_ids.astype(jnp.int32), utabt, itabt)
